# initial kernel scaffold (unmeasured)
import jax
import jax.numpy as jnp
from jax import lax
from jax.experimental import pallas as pl
from jax.experimental.pallas import tpu as pltpu

N_DEV = 32
M_PER = 128
K_PER = 128


def kernel(x, w_mat, scale_x, scale_w):
    m_tot, k_per = x.shape
    k_tot, n = w_mat.shape
    assert m_tot == N_DEV * M_PER and k_per == K_PER, (x.shape, w_mat.shape)

    def body(x_ref, w_ref, sx_ref, sw_ref, out_ref,
             send_buf, recv_buf, send_sems, recv_sems):
        my = lax.axis_index("i")

        send_buf[...] = x_ref[...].astype(jnp.float8_e4m3fn).reshape(
            N_DEV, M_PER, K_PER)

        recv_buf[my] = send_buf[my]

        rdmas = []
        for off in range(1, N_DEV):
            dst = (my + off) % N_DEV
            rdma = pltpu.make_async_remote_copy(
                src_ref=send_buf.at[dst],
                dst_ref=recv_buf.at[my],
                send_sem=send_sems.at[off],
                recv_sem=recv_sems.at[off],
                device_id=(dst,),
                device_id_type=pl.DeviceIdType.MESH,
            )
            rdma.start()
            rdmas.append(rdma)

        for r in rdmas:
            r.wait_recv()

        acc = jnp.zeros((M_PER, n), jnp.float32)
        for j in range(N_DEV):
            a = recv_buf[j]
            b = w_ref[j * K_PER:(j + 1) * K_PER, :].astype(jnp.float8_e5m2)
            acc += lax.dot_general(
                a, b, (((1,), (0,)), ((), ())),
                preferred_element_type=jnp.float32)
        out_ref[...] = acc * (sx_ref[0] * sw_ref[0])

        for r in rdmas:
            r.wait_send()

    return pl.pallas_call(
        body,
        out_shape=jax.ShapeDtypeStruct((M_PER, n), jnp.float32),
        in_specs=[
            pl.BlockSpec(memory_space=pltpu.VMEM),
            pl.BlockSpec(memory_space=pltpu.VMEM),
            pl.BlockSpec(memory_space=pltpu.SMEM),
            pl.BlockSpec(memory_space=pltpu.SMEM),
        ],
        out_specs=pl.BlockSpec(memory_space=pltpu.VMEM),
        scratch_shapes=[
            pltpu.VMEM((N_DEV, M_PER, K_PER), jnp.float8_e4m3fn),
            pltpu.VMEM((N_DEV, M_PER, K_PER), jnp.float8_e4m3fn),
            pltpu.SemaphoreType.DMA((N_DEV,)),
            pltpu.SemaphoreType.DMA((N_DEV,)),
        ],
        compiler_params=pltpu.CompilerParams(collective_id=0),
    )(x, w_mat, scale_x, scale_w)


# baseline (device time: 37083 ns/iter reference)
import jax
import jax.numpy as jnp
from jax import lax
from jax.experimental import pallas as pl
from jax.experimental.pallas import tpu as pltpu

N_DEV = 32
M_PER = 128
K_PER = 128


def kernel(x, w_mat, scale_x, scale_w):
    m_tot, k_per = x.shape
    k_tot, n = w_mat.shape
    assert m_tot == N_DEV * M_PER and k_per == K_PER, (x.shape, w_mat.shape)

    def body(x_ref, w_ref, sx_ref, sw_ref, out_ref,
             send_buf, recv_buf, send_sems, recv_sems):
        my = lax.axis_index("i")

        send_buf[...] = x_ref[...].astype(jnp.float8_e4m3fn).reshape(
            N_DEV, M_PER, K_PER)

        recv_buf[my] = send_buf[my]

        rdmas = []
        for off in range(1, N_DEV):
            dst = (my + off) % N_DEV
            rdma = pltpu.make_async_remote_copy(
                src_ref=send_buf.at[dst],
                dst_ref=recv_buf.at[my],
                send_sem=send_sems.at[off],
                recv_sem=recv_sems.at[off],
                device_id=(dst,),
                device_id_type=pl.DeviceIdType.MESH,
            )
            rdma.start()
            rdmas.append(rdma)

        for r in rdmas:
            r.wait_recv()

        acc = jnp.zeros((M_PER, n), jnp.float32)
        for j in range(N_DEV):
            a = recv_buf[j]
            b = w_ref[j * K_PER:(j + 1) * K_PER, :].astype(jnp.float8_e5m2)
            acc += lax.dot_general(
                a, b, (((1,), (0,)), ((), ())),
                preferred_element_type=jnp.float32)
        out_ref[...] = acc * (sx_ref[0] * sw_ref[0])

        for r in rdmas:
            r.wait_send()

    return pl.pallas_call(
        body,
        out_shape=jax.ShapeDtypeStruct((M_PER, n), jnp.float32),
        in_specs=[
            pl.BlockSpec(memory_space=pltpu.VMEM),
            pl.BlockSpec(memory_space=pltpu.VMEM),
            pl.BlockSpec(memory_space=pltpu.SMEM),
            pl.BlockSpec(memory_space=pltpu.SMEM),
        ],
        out_specs=pl.BlockSpec(memory_space=pltpu.VMEM),
        scratch_shapes=[
            pltpu.VMEM((N_DEV, M_PER, K_PER), jnp.float8_e4m3fn),
            pltpu.VMEM((N_DEV, M_PER, K_PER), jnp.float8_e4m3fn),
            pltpu.SemaphoreType.DMA((N_DEV,)),
            pltpu.SemaphoreType.DMA((N_DEV,)),
        ],
        compiler_params=pltpu.CompilerParams(
            vmem_limit_bytes=100 * 1024 * 1024),
    )(x, w_mat, scale_x, scale_w)


# device time: 23134 ns/iter; 1.6030x vs baseline; 1.6030x over previous
import jax
import jax.numpy as jnp
from jax import lax
from jax.experimental import pallas as pl
from jax.experimental.pallas import tpu as pltpu

N_DEV = 32
M_PER = 128
K_PER = 128
N_WCH = 8
ROWS_WCH = 4096 // N_WCH


def kernel(x, w_mat, scale_x, scale_w):
    m_tot, k_per = x.shape
    k_tot, n = w_mat.shape
    assert m_tot == N_DEV * M_PER and k_per == K_PER, (x.shape, w_mat.shape)
    assert k_tot == N_DEV * K_PER, w_mat.shape

    def body(x_ref, w_ref, sx_ref, sw_ref, out_ref,
             send_buf, recv_buf, send_sems, recv_sems, wbuf, wsems):
        my = lax.axis_index("i")

        barrier_sem = pltpu.get_barrier_semaphore()
        for off in range(1, N_DEV):
            pl.semaphore_signal(barrier_sem, inc=1,
                                device_id=((my + off) % N_DEV,),
                                device_id_type=pl.DeviceIdType.MESH)

        wcopies = []
        for c in range(N_WCH):
            cp = pltpu.make_async_copy(
                w_ref.at[pl.ds(c * ROWS_WCH, ROWS_WCH), :],
                wbuf.at[pl.ds(c * ROWS_WCH, ROWS_WCH), :],
                wsems.at[c])
            cp.start()
            wcopies.append(cp)

        send_buf[...] = x_ref[...].astype(jnp.float8_e4m3fn).reshape(
            N_DEV, M_PER, K_PER)
        recv_buf[my] = send_buf[my]

        pl.semaphore_wait(barrier_sem, N_DEV - 1)

        rdmas = []
        for off in range(1, N_DEV):
            dst = (my + off) % N_DEV
            rdma = pltpu.make_async_remote_copy(
                src_ref=send_buf.at[dst],
                dst_ref=recv_buf.at[my],
                send_sem=send_sems.at[off],
                recv_sem=recv_sems.at[off],
                device_id=(dst,),
                device_id_type=pl.DeviceIdType.MESH)
            rdma.start()
            rdmas.append(rdma)
        for r in rdmas:
            r.wait_recv()

        acc = jnp.zeros((M_PER, n), jnp.float32)
        blocks_per_ch = ROWS_WCH // K_PER
        for c in range(N_WCH):
            wcopies[c].wait()
            for jj in range(blocks_per_ch):
                j = c * blocks_per_ch + jj
                a = recv_buf[j]
                b = wbuf[j * K_PER:(j + 1) * K_PER, :].astype(
                    jnp.float8_e5m2)
                acc += lax.dot_general(a, b, (((1,), (0,)), ((), ())),
                                       preferred_element_type=jnp.float32)
        out_ref[...] = acc * (sx_ref[0] * sw_ref[0])

        for r in rdmas:
            r.wait_send()

    return pl.pallas_call(
        body,
        out_shape=jax.ShapeDtypeStruct((M_PER, n), jnp.float32),
        in_specs=[
            pl.BlockSpec(memory_space=pltpu.VMEM),
            pl.BlockSpec(memory_space=pl.ANY),
            pl.BlockSpec(memory_space=pltpu.SMEM),
            pl.BlockSpec(memory_space=pltpu.SMEM),
        ],
        out_specs=pl.BlockSpec(memory_space=pltpu.VMEM),
        scratch_shapes=[
            pltpu.VMEM((N_DEV, M_PER, K_PER), jnp.float8_e4m3fn),
            pltpu.VMEM((N_DEV, M_PER, K_PER), jnp.float8_e4m3fn),
            pltpu.SemaphoreType.DMA((N_DEV,)),
            pltpu.SemaphoreType.DMA((N_DEV,)),
            pltpu.VMEM((4096, n), jnp.float32),
            pltpu.SemaphoreType.DMA((N_WCH,)),
        ],
        compiler_params=pltpu.CompilerParams(
            vmem_limit_bytes=100 * 1024 * 1024, collective_id=0),
    )(x, w_mat, scale_x, scale_w)
